# baseline (speedup 1.0000x reference)
import functools

import jax
import jax.numpy as jnp
from jax import lax
from jax.experimental import pallas as pl
from jax.experimental.pallas import tpu as pltpu
from jax.experimental.pallas import tpu_sc as plsc

D_MODEL = 1024
N_EXPERTS = 8
N_ACTIVE = 2
CAPACITY_FACTOR = 1.25
D_FF = 4096
AUX_COEFF = 0.01

SC_CORES = 2
SC_SUBCORES = 16
SC_WORKERS = SC_CORES * SC_SUBCORES
ROW_CHUNK = 64
NBUF = 2
ROW_I32 = D_MODEL // 2

F_BLK = 1024


def _router_body(cap, n_tok, x_ref, wr_ref, dst_ref, gidx_ref, aw_ref, aux_ref,
                 x16_ref):
    n_flat = N_ACTIVE * n_tok
    x = x_ref[...]
    wr = wr_ref[...]
    logits = lax.dot_general(
        wr, x, (((1,), (1,)), ((), ())),
        preferred_element_type=jnp.float32,
    )
    m = jnp.max(logits, axis=0, keepdims=True)
    e = jnp.exp(logits - m)
    probs = e / jnp.sum(e, axis=0, keepdims=True)

    iota_e = lax.broadcasted_iota(jnp.int32, (N_EXPERTS, n_tok), 0)
    p1 = jnp.max(probs, axis=0, keepdims=True)
    i1 = jnp.min(jnp.where(probs == p1, iota_e, N_EXPERTS), axis=0, keepdims=True)
    probs2 = jnp.where(iota_e == i1, -1.0, probs)
    p2 = jnp.max(probs2, axis=0, keepdims=True)
    i2 = jnp.min(jnp.where(probs2 == p2, iota_e, N_EXPERTS), axis=0, keepdims=True)
    sw = p1 + p2 + 1e-9
    w0 = p1 / sw
    w1 = p2 / sw

    fe = jnp.concatenate([i1, i2], axis=1)
    oh = (fe == lax.broadcasted_iota(jnp.int32, (N_EXPERTS, n_flat), 0)).astype(
        jnp.float32)
    a = oh
    s = 1
    while s < n_flat:
        a = a + jnp.concatenate(
            [jnp.zeros((N_EXPERTS, s), jnp.float32), a[:, :-s]], axis=1)
        s *= 2
    excl = a - oh
    rank = jnp.sum(oh * excl, axis=0, keepdims=True).astype(jnp.int32)
    acc = rank < cap

    flat_w = jnp.concatenate([w0, w1], axis=1)
    trash = N_EXPERTS * cap
    dst_ref[...] = jnp.where(acc, fe * cap + rank, trash)
    gidx_ref[...] = fe * cap + jnp.minimum(rank, cap - 1)
    aw_ref[...] = jnp.where(acc, flat_w, 0.0)

    tot_e = a[:, n_flat - 1:n_flat]
    cnt = jnp.minimum(tot_e, float(cap))
    total = jnp.maximum(jnp.sum(cnt), 1.0)
    f_i = cnt / total
    p_mean = jnp.sum(probs, axis=1, keepdims=True) / float(n_tok)
    aux = AUX_COEFF * N_EXPERTS * jnp.sum(f_i * p_mean)
    aux_ref[...] = jnp.reshape(aux, (1, 1))
    x16_ref[...] = x.astype(jnp.bfloat16)


def _ffn_body(n_j, x_ref, gw_ref, uw_ref, dw_ref, y_ref, acc_ref):
    j = pl.program_id(1)
    xb = x_ref[...]
    gw = gw_ref[0].astype(jnp.bfloat16)
    uw = uw_ref[0].astype(jnp.bfloat16)
    dw = dw_ref[0].astype(jnp.bfloat16)
    dn = (((1,), (1,)), ((), ()))
    g = lax.dot_general(xb, gw, dn, preferred_element_type=jnp.float32)
    u = lax.dot_general(xb, uw, dn, preferred_element_type=jnp.float32)
    h = (g * (1.0 / (1.0 + jnp.exp(-g)))) * u
    y = lax.dot_general(h.astype(jnp.bfloat16), dw, dn,
                        preferred_element_type=jnp.float32)

    @pl.when(j == 0)
    def _():
        acc_ref[...] = y

    @pl.when(j != 0)
    def _():
        acc_ref[...] = acc_ref[...] + y

    @pl.when(j == n_j - 1)
    def _():
        y_ref[0] = acc_ref[...].astype(jnp.bfloat16)


def _final_body(y0_ref, y1_ref, aw0_ref, aw1_ref, o_ref):
    aw0 = aw0_ref[...]
    aw1 = aw1_ref[...]
    y0 = y0_ref[...].astype(jnp.float32)
    y1 = y1_ref[...].astype(jnp.float32)
    o_ref[...] = (jnp.where(aw0 > 0, aw0 * y0, 0.0)
                  + jnp.where(aw1 > 0, aw1 * y1, 0.0))


def _sc_mesh():
    return plsc.VectorSubcoreMesh(
        core_axis_name="c", subcore_axis_name="s",
        num_cores=SC_CORES, num_subcores=SC_SUBCORES)


def _make_dispatch(n_tok, n_rows):
    n_flat = N_ACTIVE * n_tok
    per_w = n_flat // SC_WORKERS
    n_ch = per_w // ROW_CHUNK

    @functools.partial(
        pl.kernel,
        out_type=jax.ShapeDtypeStruct((n_rows, ROW_I32), jnp.int32),
        mesh=_sc_mesh(),
        scratch_types=[
            pltpu.VMEM((NBUF, ROW_CHUNK), jnp.int32),
            pltpu.VMEM((NBUF, ROW_CHUNK, ROW_I32), jnp.int32),
            [pltpu.SemaphoreType.DMA] * NBUF,
        ],
    )
    def dispatch(x_hbm, dst_hbm, out_hbm, idx_v, rows_v, sems):
        wid = lax.axis_index("s") * SC_CORES + lax.axis_index("c")
        base = wid * per_w

        def load(i, b):
            off = base + i * ROW_CHUNK
            src = lax.rem(off, n_tok)
            pltpu.sync_copy(dst_hbm.at[pl.ds(off, ROW_CHUNK)], idx_v.at[b])
            pltpu.sync_copy(x_hbm.at[pl.ds(src, ROW_CHUNK)], rows_v.at[b])

        def scatter(b):
            return pltpu.async_copy(rows_v.at[b], out_hbm.at[idx_v.at[b]],
                                    sems[b])

        load(0, 0)
        cp = scatter(0)
        for i in range(1, n_ch):
            b = i % NBUF
            load(i, b)
            cp.wait()
            cp = scatter(b)
        cp.wait()

    return dispatch


def _make_combine(n_tok, n_rows):
    n_flat = N_ACTIVE * n_tok
    per_w = n_flat // SC_WORKERS
    n_ch = per_w // ROW_CHUNK

    @functools.partial(
        pl.kernel,
        out_type=jax.ShapeDtypeStruct((n_flat, ROW_I32), jnp.int32),
        mesh=_sc_mesh(),
        scratch_types=[
            pltpu.VMEM((NBUF, ROW_CHUNK), jnp.int32),
            pltpu.VMEM((NBUF, ROW_CHUNK, ROW_I32), jnp.int32),
            [pltpu.SemaphoreType.DMA] * NBUF,
        ],
    )
    def combine(y_hbm, gidx_hbm, out_hbm, idx_v, rows_v, sems):
        wid = lax.axis_index("s") * SC_CORES + lax.axis_index("c")
        base = wid * per_w

        def loadidx(i, b):
            off = base + i * ROW_CHUNK
            pltpu.sync_copy(gidx_hbm.at[pl.ds(off, ROW_CHUNK)], idx_v.at[b])

        def gather(b):
            return pltpu.async_copy(y_hbm.at[idx_v.at[b]], rows_v.at[b],
                                    sems[b])

        def store(i, b):
            off = base + i * ROW_CHUNK
            pltpu.sync_copy(rows_v.at[b], out_hbm.at[pl.ds(off, ROW_CHUNK)])

        loadidx(0, 0)
        cp = gather(0)
        for i in range(1, n_ch):
            b = i % NBUF
            loadidx(i, b)
            cp.wait()
            cp = gather(b)
            store(i - 1, (i - 1) % NBUF)
        cp.wait()
        store(n_ch - 1, (n_ch - 1) % NBUF)

    return combine


def kernel(x, W_router, gate_w, up_w, down_w):
    bb, tt, d = x.shape
    n_tok = bb * tt
    n_flat = N_ACTIVE * n_tok
    cap = max(int(tt * N_ACTIVE * CAPACITY_FACTOR / N_EXPERTS), 1)
    n_rows = N_EXPERTS * cap + 8

    x_flat = x.reshape(n_tok, d)

    dst, gidx, aw, aux, x16 = pl.pallas_call(
        functools.partial(_router_body, cap, n_tok),
        out_shape=[
            jax.ShapeDtypeStruct((1, n_flat), jnp.int32),
            jax.ShapeDtypeStruct((1, n_flat), jnp.int32),
            jax.ShapeDtypeStruct((1, n_flat), jnp.float32),
            jax.ShapeDtypeStruct((1, 1), jnp.float32),
            jax.ShapeDtypeStruct((n_tok, d), jnp.bfloat16),
        ],
    )(x_flat, W_router)

    x16i = lax.bitcast_convert_type(
        x16.reshape(n_tok, d // 2, 2), jnp.int32)

    bufi = _make_dispatch(n_tok, n_rows)(x16i, dst.reshape(n_flat))
    buf = lax.bitcast_convert_type(bufi, jnp.bfloat16).reshape(n_rows, d)

    n_j = D_FF // F_BLK
    y = pl.pallas_call(
        functools.partial(_ffn_body, n_j),
        grid=(N_EXPERTS, n_j),
        in_specs=[
            pl.BlockSpec((cap, d), lambda e, j: (e, 0)),
            pl.BlockSpec((1, F_BLK, d), lambda e, j: (e, j, 0)),
            pl.BlockSpec((1, F_BLK, d), lambda e, j: (e, j, 0)),
            pl.BlockSpec((1, d, F_BLK), lambda e, j: (e, 0, j)),
        ],
        out_specs=pl.BlockSpec((1, cap, d), lambda e, j: (e, 0, 0)),
        out_shape=jax.ShapeDtypeStruct((N_EXPERTS, cap, d), jnp.bfloat16),
        scratch_shapes=[pltpu.VMEM((cap, d), jnp.float32)],
        compiler_params=pltpu.CompilerParams(
            dimension_semantics=("parallel", "arbitrary")),
    )(buf, gate_w, up_w, down_w)

    yi = lax.bitcast_convert_type(
        y.reshape(N_EXPERTS * cap, d // 2, 2), jnp.int32)
    ygi = _make_combine(n_tok, N_EXPERTS * cap)(yi, gidx.reshape(n_flat))
    yg = lax.bitcast_convert_type(ygi, jnp.bfloat16).reshape(n_flat, d)

    aw_col = aw.reshape(n_flat, 1)
    blk = 512
    out_flat = pl.pallas_call(
        _final_body,
        grid=(n_tok // blk,),
        in_specs=[
            pl.BlockSpec((blk, d), lambda i: (i, 0)),
            pl.BlockSpec((blk, d), lambda i, n=n_tok // blk: (i + n, 0)),
            pl.BlockSpec((blk, 1), lambda i: (i, 0)),
            pl.BlockSpec((blk, 1), lambda i, n=n_tok // blk: (i + n, 0)),
        ],
        out_specs=pl.BlockSpec((blk, d), lambda i: (i, 0)),
        out_shape=jax.ShapeDtypeStruct((n_tok, d), jnp.float32),
        compiler_params=pltpu.CompilerParams(
            dimension_semantics=("parallel",)),
    )(yg, yg, aw_col, aw_col)

    return out_flat.reshape(bb, tt, d), aux.reshape(())


# speedup vs baseline: 2.1982x; 2.1982x over previous
import functools

import jax
import jax.numpy as jnp
from jax import lax
from jax.experimental import pallas as pl
from jax.experimental.pallas import tpu as pltpu
from jax.experimental.pallas import tpu_sc as plsc

D_MODEL = 1024
N_EXPERTS = 8
N_ACTIVE = 2
CAPACITY_FACTOR = 1.25
D_FF = 4096
AUX_COEFF = 0.01

SC_CORES = 2
SC_SUBCORES = 16
SC_WORKERS = SC_CORES * SC_SUBCORES
ROW_CHUNK = 64
NBUF = 2
ROW_I32 = D_MODEL // 2


def _pack_halves(v16):
    n, dd = v16.shape
    bits = lax.convert_element_type(
        lax.bitcast_convert_type(v16, jnp.uint16), jnp.uint32)
    w = bits[:, :dd // 2] | (bits[:, dd // 2:] << 16)
    return lax.bitcast_convert_type(w, jnp.int32)


def _unpack_halves(w32):
    w = lax.bitcast_convert_type(w32, jnp.uint32)
    lo = lax.bitcast_convert_type(
        lax.convert_element_type(w & 0xFFFF, jnp.uint16), jnp.bfloat16)
    hi = lax.bitcast_convert_type(
        lax.convert_element_type(w >> 16, jnp.uint16), jnp.bfloat16)
    return jnp.concatenate([lo, hi], axis=1)

F_BLK = 1024


def _router_body(cap, n_tok, x_ref, wr_ref, dst_ref, gidx_ref, aw_ref, aux_ref,
                 x16_ref):
    n_flat = N_ACTIVE * n_tok
    x = x_ref[...]
    wr = wr_ref[...]
    logits = lax.dot_general(
        wr, x, (((1,), (1,)), ((), ())),
        preferred_element_type=jnp.float32,
    )
    m = jnp.max(logits, axis=0, keepdims=True)
    e = jnp.exp(logits - m)
    probs = e / jnp.sum(e, axis=0, keepdims=True)

    iota_e = lax.broadcasted_iota(jnp.int32, (N_EXPERTS, n_tok), 0)
    p1 = jnp.max(probs, axis=0, keepdims=True)
    i1 = jnp.min(jnp.where(probs == p1, iota_e, N_EXPERTS), axis=0, keepdims=True)
    probs2 = jnp.where(iota_e == i1, -1.0, probs)
    p2 = jnp.max(probs2, axis=0, keepdims=True)
    i2 = jnp.min(jnp.where(probs2 == p2, iota_e, N_EXPERTS), axis=0, keepdims=True)
    sw = p1 + p2 + 1e-9
    w0 = p1 / sw
    w1 = p2 / sw

    fe = jnp.concatenate([i1, i2], axis=1)
    oh = (fe == lax.broadcasted_iota(jnp.int32, (N_EXPERTS, n_flat), 0)).astype(
        jnp.float32)
    a = oh
    s = 1
    while s < n_flat:
        a = a + jnp.concatenate(
            [jnp.zeros((N_EXPERTS, s), jnp.float32), a[:, :-s]], axis=1)
        s *= 2
    excl = a - oh
    rank = jnp.sum(oh * excl, axis=0, keepdims=True).astype(jnp.int32)
    acc = rank < cap

    flat_w = jnp.concatenate([w0, w1], axis=1)
    trash = N_EXPERTS * cap
    dst_ref[...] = jnp.where(acc, fe * cap + rank, trash)
    gidx_ref[...] = fe * cap + jnp.minimum(rank, cap - 1)
    aw_ref[...] = jnp.where(acc, flat_w, 0.0)

    tot_e = a[:, n_flat - 1:n_flat]
    cnt = jnp.minimum(tot_e, float(cap))
    total = jnp.maximum(jnp.sum(cnt), 1.0)
    f_i = cnt / total
    p_mean = jnp.sum(probs, axis=1, keepdims=True) / float(n_tok)
    aux = AUX_COEFF * N_EXPERTS * jnp.sum(f_i * p_mean)
    aux_ref[...] = jnp.reshape(aux, (1, 1))
    x16_ref[...] = _pack_halves(x.astype(jnp.bfloat16))


def _ffn_body(n_j, x_ref, gw_ref, uw_ref, dw_ref, y_ref, acc_ref):
    j = pl.program_id(1)
    xb = _unpack_halves(x_ref[...])
    gw = gw_ref[0].astype(jnp.bfloat16)
    uw = uw_ref[0].astype(jnp.bfloat16)
    dw = dw_ref[0].astype(jnp.bfloat16)
    dn = (((1,), (1,)), ((), ()))
    g = lax.dot_general(xb, gw, dn, preferred_element_type=jnp.float32)
    u = lax.dot_general(xb, uw, dn, preferred_element_type=jnp.float32)
    h = (g * (1.0 / (1.0 + jnp.exp(-g)))) * u
    y = lax.dot_general(h.astype(jnp.bfloat16), dw, dn,
                        preferred_element_type=jnp.float32)

    @pl.when(j == 0)
    def _():
        acc_ref[...] = y

    @pl.when(j != 0)
    def _():
        acc_ref[...] = acc_ref[...] + y

    @pl.when(j == n_j - 1)
    def _():
        y_ref[0] = _pack_halves(acc_ref[...].astype(jnp.bfloat16))


def _final_body(y0_ref, y1_ref, aw0_ref, aw1_ref, o_ref):
    aw0 = aw0_ref[...]
    aw1 = aw1_ref[...]
    y0 = _unpack_halves(y0_ref[...]).astype(jnp.float32)
    y1 = _unpack_halves(y1_ref[...]).astype(jnp.float32)
    o_ref[...] = (jnp.where(aw0 > 0, aw0 * y0, 0.0)
                  + jnp.where(aw1 > 0, aw1 * y1, 0.0))


def _sc_mesh():
    return plsc.VectorSubcoreMesh(
        core_axis_name="c", subcore_axis_name="s",
        num_cores=SC_CORES, num_subcores=SC_SUBCORES)


def _make_dispatch(n_tok, n_rows):
    n_flat = N_ACTIVE * n_tok
    per_w = n_flat // SC_WORKERS
    n_ch = per_w // ROW_CHUNK

    @functools.partial(
        pl.kernel,
        out_type=jax.ShapeDtypeStruct((n_rows, ROW_I32), jnp.int32),
        mesh=_sc_mesh(),
        scratch_types=[
            pltpu.VMEM((NBUF, ROW_CHUNK), jnp.int32),
            pltpu.VMEM((NBUF, ROW_CHUNK, ROW_I32), jnp.int32),
            [pltpu.SemaphoreType.DMA] * NBUF,
        ],
    )
    def dispatch(x_hbm, dst_hbm, out_hbm, idx_v, rows_v, sems):
        wid = lax.axis_index("s") * SC_CORES + lax.axis_index("c")
        base = wid * per_w

        def load(i, b):
            off = base + i * ROW_CHUNK
            src = lax.rem(off, n_tok)
            pltpu.sync_copy(dst_hbm.at[pl.ds(off, ROW_CHUNK)], idx_v.at[b])
            pltpu.sync_copy(x_hbm.at[pl.ds(src, ROW_CHUNK)], rows_v.at[b])

        def scatter(b):
            return pltpu.async_copy(rows_v.at[b], out_hbm.at[idx_v.at[b]],
                                    sems[b])

        load(0, 0)
        cp = scatter(0)
        for i in range(1, n_ch):
            b = i % NBUF
            load(i, b)
            cp.wait()
            cp = scatter(b)
        cp.wait()

    return dispatch


def _make_combine(n_tok, n_rows):
    n_flat = N_ACTIVE * n_tok
    per_w = n_flat // SC_WORKERS
    n_ch = per_w // ROW_CHUNK

    @functools.partial(
        pl.kernel,
        out_type=jax.ShapeDtypeStruct((n_flat, ROW_I32), jnp.int32),
        mesh=_sc_mesh(),
        scratch_types=[
            pltpu.VMEM((NBUF, ROW_CHUNK), jnp.int32),
            pltpu.VMEM((NBUF, ROW_CHUNK, ROW_I32), jnp.int32),
            [pltpu.SemaphoreType.DMA] * NBUF,
        ],
    )
    def combine(y_hbm, gidx_hbm, out_hbm, idx_v, rows_v, sems):
        wid = lax.axis_index("s") * SC_CORES + lax.axis_index("c")
        base = wid * per_w

        def loadidx(i, b):
            off = base + i * ROW_CHUNK
            pltpu.sync_copy(gidx_hbm.at[pl.ds(off, ROW_CHUNK)], idx_v.at[b])

        def gather(b):
            return pltpu.async_copy(y_hbm.at[idx_v.at[b]], rows_v.at[b],
                                    sems[b])

        def store(i, b):
            off = base + i * ROW_CHUNK
            pltpu.sync_copy(rows_v.at[b], out_hbm.at[pl.ds(off, ROW_CHUNK)])

        loadidx(0, 0)
        cp = gather(0)
        for i in range(1, n_ch):
            b = i % NBUF
            loadidx(i, b)
            cp.wait()
            cp = gather(b)
            store(i - 1, (i - 1) % NBUF)
        cp.wait()
        store(n_ch - 1, (n_ch - 1) % NBUF)

    return combine


def kernel(x, W_router, gate_w, up_w, down_w):
    bb, tt, d = x.shape
    n_tok = bb * tt
    n_flat = N_ACTIVE * n_tok
    cap = max(int(tt * N_ACTIVE * CAPACITY_FACTOR / N_EXPERTS), 1)
    n_rows = N_EXPERTS * cap + 8

    x_flat = x.reshape(n_tok, d)

    dst, gidx, aw, aux, x16 = pl.pallas_call(
        functools.partial(_router_body, cap, n_tok),
        out_shape=[
            jax.ShapeDtypeStruct((1, n_flat), jnp.int32),
            jax.ShapeDtypeStruct((1, n_flat), jnp.int32),
            jax.ShapeDtypeStruct((1, n_flat), jnp.float32),
            jax.ShapeDtypeStruct((1, 1), jnp.float32),
            jax.ShapeDtypeStruct((n_tok, d // 2), jnp.int32),
        ],
    )(x_flat, W_router)

    buf = _make_dispatch(n_tok, n_rows)(x16, dst.reshape(n_flat))

    n_j = D_FF // F_BLK
    y = pl.pallas_call(
        functools.partial(_ffn_body, n_j),
        grid=(N_EXPERTS, n_j),
        in_specs=[
            pl.BlockSpec((cap, d // 2), lambda e, j: (e, 0)),
            pl.BlockSpec((1, F_BLK, d), lambda e, j: (e, j, 0)),
            pl.BlockSpec((1, F_BLK, d), lambda e, j: (e, j, 0)),
            pl.BlockSpec((1, d, F_BLK), lambda e, j: (e, 0, j)),
        ],
        out_specs=pl.BlockSpec((1, cap, d // 2), lambda e, j: (e, 0, 0)),
        out_shape=jax.ShapeDtypeStruct((N_EXPERTS, cap, d // 2), jnp.int32),
        scratch_shapes=[pltpu.VMEM((cap, d), jnp.float32)],
        compiler_params=pltpu.CompilerParams(
            dimension_semantics=("parallel", "arbitrary")),
    )(buf, gate_w, up_w, down_w)

    yg = _make_combine(n_tok, N_EXPERTS * cap)(
        y.reshape(N_EXPERTS * cap, d // 2), gidx.reshape(n_flat))

    aw_col = aw.reshape(n_flat, 1)
    blk = 512
    out_flat = pl.pallas_call(
        _final_body,
        grid=(n_tok // blk,),
        in_specs=[
            pl.BlockSpec((blk, d // 2), lambda i: (i, 0)),
            pl.BlockSpec((blk, d // 2), lambda i, n=n_tok // blk: (i + n, 0)),
            pl.BlockSpec((blk, 1), lambda i: (i, 0)),
            pl.BlockSpec((blk, 1), lambda i, n=n_tok // blk: (i + n, 0)),
        ],
        out_specs=pl.BlockSpec((blk, d), lambda i: (i, 0)),
        out_shape=jax.ShapeDtypeStruct((n_tok, d), jnp.float32),
        compiler_params=pltpu.CompilerParams(
            dimension_semantics=("parallel",)),
    )(yg, yg, aw_col, aw_col)

    return out_flat.reshape(bb, tt, d), aux.reshape(())


# speedup vs baseline: 2.2666x; 1.0311x over previous
import functools

import jax
import jax.numpy as jnp
from jax import lax
from jax.experimental import pallas as pl
from jax.experimental.pallas import tpu as pltpu
from jax.experimental.pallas import tpu_sc as plsc

D_MODEL = 1024
N_EXPERTS = 8
N_ACTIVE = 2
CAPACITY_FACTOR = 1.25
D_FF = 4096
AUX_COEFF = 0.01

SC_CORES = 2
SC_SUBCORES = 16
SC_WORKERS = SC_CORES * SC_SUBCORES
NBUF = 2

HALF = D_MODEL // 2
ROW_W = HALF + 128
DISP_CHUNK = 64
F_BLK = 1024


def _pack_halves(v16):
    n, dd = v16.shape
    bits = lax.convert_element_type(
        lax.bitcast_convert_type(v16, jnp.uint16), jnp.uint32)
    w = bits[:, :dd // 2] | (bits[:, dd // 2:] << 16)
    return lax.bitcast_convert_type(w, jnp.int32)


def _unpack_halves(w32):
    w = lax.bitcast_convert_type(w32, jnp.uint32)
    lo = lax.bitcast_convert_type(
        lax.convert_element_type(w & 0xFFFF, jnp.uint16), jnp.bfloat16)
    hi = lax.bitcast_convert_type(
        lax.convert_element_type(w >> 16, jnp.uint16), jnp.bfloat16)
    return jnp.concatenate([lo, hi], axis=1)


def _router_body(cap, n_tok, x_ref, wr_ref, dst_ref, aw_ref, cnt_ref, aux_ref,
                 xp_ref):
    n_flat = N_ACTIVE * n_tok
    x = x_ref[...]
    wr = wr_ref[...]
    logits = lax.dot_general(
        wr, x, (((1,), (1,)), ((), ())),
        preferred_element_type=jnp.float32,
    )
    m = jnp.max(logits, axis=0, keepdims=True)
    e = jnp.exp(logits - m)
    probs = e / jnp.sum(e, axis=0, keepdims=True)

    iota_e = lax.broadcasted_iota(jnp.int32, (N_EXPERTS, n_tok), 0)
    p1 = jnp.max(probs, axis=0, keepdims=True)
    i1 = jnp.min(jnp.where(probs == p1, iota_e, N_EXPERTS), axis=0, keepdims=True)
    probs2 = jnp.where(iota_e == i1, -1.0, probs)
    p2 = jnp.max(probs2, axis=0, keepdims=True)
    i2 = jnp.min(jnp.where(probs2 == p2, iota_e, N_EXPERTS), axis=0, keepdims=True)
    sw = p1 + p2 + 1e-9
    w0 = p1 / sw
    w1 = p2 / sw

    fe = jnp.concatenate([i1, i2], axis=1)
    oh = (fe == lax.broadcasted_iota(jnp.int32, (N_EXPERTS, n_flat), 0)).astype(
        jnp.float32)
    a = oh
    s = 1
    while s < n_flat:
        a = a + jnp.concatenate(
            [jnp.zeros((N_EXPERTS, s), jnp.float32), a[:, :-s]], axis=1)
        s *= 2
    excl = a - oh
    rank = jnp.sum(oh * excl, axis=0, keepdims=True).astype(jnp.int32)
    acc = rank < cap

    flat_w = jnp.concatenate([w0, w1], axis=1)
    trash = N_EXPERTS * cap
    dst_ref[...] = jnp.where(acc, fe * cap + rank, trash)
    aw_ref[...] = jnp.where(acc, flat_w, 0.0)

    tot_e = a[:, n_flat - 1:n_flat]
    cnt = jnp.minimum(tot_e, float(cap))
    cnt_row = lax.dot_general(
        jnp.ones((1, 1), jnp.float32), cnt, (((1,), (1,)), ((), ())),
        preferred_element_type=jnp.float32)
    cnt_ref[...] = cnt_row.astype(jnp.int32)

    total = jnp.maximum(jnp.sum(cnt), 1.0)
    f_i = cnt / total
    p_mean = jnp.sum(probs, axis=1, keepdims=True) / float(n_tok)
    aux = AUX_COEFF * N_EXPERTS * jnp.sum(f_i * p_mean)
    aux_ref[...] = jnp.reshape(aux, (1, 1))

    xp = _pack_halves(x.astype(jnp.bfloat16))
    tcol = lax.broadcasted_iota(jnp.int32, (n_tok, 8), 0)
    xp_ref[pl.ds(0, n_tok), pl.ds(0, HALF)] = xp
    xp_ref[pl.ds(0, n_tok), pl.ds(HALF, 8)] = tcol
    xp_ref[pl.ds(n_tok, n_tok), pl.ds(0, HALF)] = xp
    xp_ref[pl.ds(n_tok, n_tok), pl.ds(HALF, 8)] = tcol + n_tok


def _ffn_body(n_j, cap, trash, x_ref, gw_ref, uw_ref, dw_ref, cnt_ref, y_ref,
              acc_ref):
    e = pl.program_id(0)
    j = pl.program_id(1)
    xb = _unpack_halves(x_ref[:, :HALF])
    gw = gw_ref[0].astype(jnp.bfloat16)
    uw = uw_ref[0].astype(jnp.bfloat16)
    dw = dw_ref[0].astype(jnp.bfloat16)
    dn = (((1,), (1,)), ((), ()))
    g = lax.dot_general(xb, gw, dn, preferred_element_type=jnp.float32)
    u = lax.dot_general(xb, uw, dn, preferred_element_type=jnp.float32)
    h = (g * (1.0 / (1.0 + jnp.exp(-g)))) * u
    y = lax.dot_general(h.astype(jnp.bfloat16), dw, dn,
                        preferred_element_type=jnp.float32)

    @pl.when(j == 0)
    def _():
        acc_ref[...] = y

    @pl.when(j != 0)
    def _():
        acc_ref[...] = acc_ref[...] + y

    @pl.when(j == n_j - 1)
    def _():
        py = _pack_halves(acc_ref[...].astype(jnp.bfloat16))
        dcol = x_ref[:, HALF:HALF + 1]
        rowc = lax.broadcasted_iota(jnp.int32, (cap, 1), 0)
        dcol = jnp.where(rowc < cnt_ref[0, e], dcol, trash)
        y_ref[0] = jnp.concatenate(
            [py, jnp.broadcast_to(dcol, (cap, ROW_W - HALF))], axis=1)


def _final_body(y0_ref, y1_ref, aw0_ref, aw1_ref, o_ref):
    aw0 = aw0_ref[...]
    aw1 = aw1_ref[...]
    y0 = _unpack_halves(y0_ref[:, :HALF]).astype(jnp.float32)
    y1 = _unpack_halves(y1_ref[:, :HALF]).astype(jnp.float32)
    o_ref[...] = (jnp.where(aw0 > 0, aw0 * y0, 0.0)
                  + jnp.where(aw1 > 0, aw1 * y1, 0.0))


def _sc_mesh():
    return plsc.VectorSubcoreMesh(
        core_axis_name="c", subcore_axis_name="s",
        num_cores=SC_CORES, num_subcores=SC_SUBCORES)


def _make_dispatch(n_flat, n_rows):
    per_w = n_flat // SC_WORKERS
    n_ch = per_w // DISP_CHUNK

    @functools.partial(
        pl.kernel,
        out_type=jax.ShapeDtypeStruct((n_rows, ROW_W), jnp.int32),
        mesh=_sc_mesh(),
        scratch_types=[
            pltpu.VMEM((NBUF, DISP_CHUNK), jnp.int32),
            pltpu.VMEM((NBUF, DISP_CHUNK, ROW_W), jnp.int32),
            [pltpu.SemaphoreType.DMA] * NBUF,
        ],
    )
    def dispatch(xp_hbm, dst_hbm, out_hbm, idx_v, rows_v, sems):
        wid = lax.axis_index("s") * SC_CORES + lax.axis_index("c")
        base = wid * per_w

        def load(i, b):
            off = base + i * DISP_CHUNK
            pltpu.sync_copy(dst_hbm.at[pl.ds(off, DISP_CHUNK)], idx_v.at[b])
            pltpu.sync_copy(xp_hbm.at[pl.ds(off, DISP_CHUNK)], rows_v.at[b])

        def scatter(b):
            return pltpu.async_copy(rows_v.at[b], out_hbm.at[idx_v.at[b]],
                                    sems[b])

        load(0, 0)
        cp = scatter(0)
        for i in range(1, n_ch):
            b = i % NBUF
            load(i, b)
            cp.wait()
            cp = scatter(b)
        cp.wait()

    return dispatch


def _make_combine(n_slots, n_out):
    per_w = n_slots // SC_WORKERS
    chunk = per_w // NBUF
    assert per_w % NBUF == 0 and chunk % 8 == 0

    @functools.partial(
        pl.kernel,
        out_type=jax.ShapeDtypeStruct((n_out, ROW_W), jnp.int32),
        mesh=_sc_mesh(),
        scratch_types=[
            pltpu.VMEM((NBUF, chunk), jnp.int32),
            pltpu.VMEM((NBUF, chunk, ROW_W), jnp.int32),
            [pltpu.SemaphoreType.DMA] * NBUF,
        ],
        compiler_params=pltpu.CompilerParams(needs_layout_passes=False),
    )
    def combine(y_hbm, out_hbm, idx_v, rows_v, sems):
        wid = lax.axis_index("s") * SC_CORES + lax.axis_index("c")
        base = wid * per_w
        lane = jnp.full((16,), HALF, jnp.int32)

        def load(i, b):
            off = base + i * chunk
            pltpu.sync_copy(y_hbm.at[pl.ds(off, chunk)], rows_v.at[b])
            for j in range(chunk // 16):
                rows16 = lax.iota(jnp.int32, 16) + 16 * j
                vals = plsc.load_gather(rows_v.at[b], [rows16, lane])
                idx_v[b, pl.ds(16 * j, 16)] = vals

        def scatter(b):
            return pltpu.async_copy(rows_v.at[b], out_hbm.at[idx_v.at[b]],
                                    sems[b])

        load(0, 0)
        cp = scatter(0)
        for i in range(1, NBUF):
            load(i, i)
            cp.wait()
            cp = scatter(i)
        cp.wait()

    return combine


def kernel(x, W_router, gate_w, up_w, down_w):
    bb, tt, d = x.shape
    n_tok = bb * tt
    n_flat = N_ACTIVE * n_tok
    cap = max(int(tt * N_ACTIVE * CAPACITY_FACTOR / N_EXPERTS), 1)
    n_rows = N_EXPERTS * cap + 8
    n_out = n_flat + 8

    x_flat = x.reshape(n_tok, d)

    dst, aw, cnt, aux, xp = pl.pallas_call(
        functools.partial(_router_body, cap, n_tok),
        out_shape=[
            jax.ShapeDtypeStruct((1, n_flat), jnp.int32),
            jax.ShapeDtypeStruct((1, n_flat), jnp.float32),
            jax.ShapeDtypeStruct((1, N_EXPERTS), jnp.int32),
            jax.ShapeDtypeStruct((1, 1), jnp.float32),
            jax.ShapeDtypeStruct((n_flat, ROW_W), jnp.int32),
        ],
    )(x_flat, W_router)

    buf = _make_dispatch(n_flat, n_rows)(xp, dst.reshape(n_flat))

    n_j = D_FF // F_BLK
    y = pl.pallas_call(
        functools.partial(_ffn_body, n_j, cap, n_flat),
        grid=(N_EXPERTS, n_j),
        in_specs=[
            pl.BlockSpec((cap, ROW_W), lambda e, j: (e, 0)),
            pl.BlockSpec((1, F_BLK, d), lambda e, j: (e, j, 0)),
            pl.BlockSpec((1, F_BLK, d), lambda e, j: (e, j, 0)),
            pl.BlockSpec((1, d, F_BLK), lambda e, j: (e, 0, j)),
            pl.BlockSpec(memory_space=pltpu.SMEM),
        ],
        out_specs=pl.BlockSpec((1, cap, ROW_W), lambda e, j: (e, 0, 0)),
        out_shape=jax.ShapeDtypeStruct((N_EXPERTS, cap, ROW_W), jnp.int32),
        scratch_shapes=[pltpu.VMEM((cap, d), jnp.float32)],
        compiler_params=pltpu.CompilerParams(
            dimension_semantics=("parallel", "arbitrary")),
    )(buf, gate_w, up_w, down_w, cnt)

    yg = _make_combine(N_EXPERTS * cap, n_out)(
        y.reshape(N_EXPERTS * cap, ROW_W))

    aw_col = aw.reshape(n_flat, 1)
    blk = 512
    out_flat = pl.pallas_call(
        _final_body,
        grid=(n_tok // blk,),
        in_specs=[
            pl.BlockSpec((blk, ROW_W), lambda i: (i, 0)),
            pl.BlockSpec((blk, ROW_W), lambda i, n=n_tok // blk: (i + n, 0)),
            pl.BlockSpec((blk, 1), lambda i: (i, 0)),
            pl.BlockSpec((blk, 1), lambda i, n=n_tok // blk: (i + n, 0)),
        ],
        out_specs=pl.BlockSpec((blk, d), lambda i: (i, 0)),
        out_shape=jax.ShapeDtypeStruct((n_tok, d), jnp.float32),
        compiler_params=pltpu.CompilerParams(
            dimension_semantics=("parallel",)),
    )(yg, yg, aw_col, aw_col)

    return out_flat.reshape(bb, tt, d), aux.reshape(())


# speedup vs baseline: 3.1648x; 1.3963x over previous
import functools

import jax
import jax.numpy as jnp
from jax import lax
from jax.experimental import pallas as pl
from jax.experimental.pallas import tpu as pltpu
from jax.experimental.pallas import tpu_sc as plsc

D_MODEL = 1024
N_EXPERTS = 8
N_ACTIVE = 2
CAPACITY_FACTOR = 1.25
D_FF = 4096
AUX_COEFF = 0.01

SC_CORES = 2
SC_SUBCORES = 16
SC_WORKERS = SC_CORES * SC_SUBCORES
NBUF = 2

HALF = D_MODEL // 2
ROW_W = HALF + 128
DISP_CHUNK = 64
F_BLK = 1024


def _pack_halves(v16):
    n, dd = v16.shape
    bits = lax.convert_element_type(
        lax.bitcast_convert_type(v16, jnp.uint16), jnp.uint32)
    w = bits[:, :dd // 2] | (bits[:, dd // 2:] << 16)
    return lax.bitcast_convert_type(w, jnp.int32)


def _unpack_halves(w32):
    w = lax.bitcast_convert_type(w32, jnp.uint32)
    lo = lax.bitcast_convert_type(
        lax.convert_element_type(w & 0xFFFF, jnp.uint16), jnp.bfloat16)
    hi = lax.bitcast_convert_type(
        lax.convert_element_type(w >> 16, jnp.uint16), jnp.bfloat16)
    return jnp.concatenate([lo, hi], axis=1)


def _router_body(cap, n_tok, x_ref, wr_ref, dst_ref, aw_ref, cnt_ref, aux_ref,
                 xp_ref):
    n_flat = N_ACTIVE * n_tok
    x = x_ref[...]
    wr = wr_ref[...]
    logits = lax.dot_general(
        wr, x, (((1,), (1,)), ((), ())),
        preferred_element_type=jnp.float32,
    )
    m = jnp.max(logits, axis=0, keepdims=True)
    e = jnp.exp(logits - m)
    probs = e / jnp.sum(e, axis=0, keepdims=True)

    iota_e = lax.broadcasted_iota(jnp.int32, (N_EXPERTS, n_tok), 0)
    p1 = jnp.max(probs, axis=0, keepdims=True)
    i1 = jnp.min(jnp.where(probs == p1, iota_e, N_EXPERTS), axis=0, keepdims=True)
    probs2 = jnp.where(iota_e == i1, -1.0, probs)
    p2 = jnp.max(probs2, axis=0, keepdims=True)
    i2 = jnp.min(jnp.where(probs2 == p2, iota_e, N_EXPERTS), axis=0, keepdims=True)
    sw = p1 + p2 + 1e-9
    w0 = p1 / sw
    w1 = p2 / sw

    fe = jnp.concatenate([i1, i2], axis=1)
    oh = (fe == lax.broadcasted_iota(jnp.int32, (N_EXPERTS, n_flat), 0)).astype(
        jnp.float32)
    a = oh
    s = 1
    while s < n_flat:
        a = a + jnp.concatenate(
            [jnp.zeros((N_EXPERTS, s), jnp.float32), a[:, :-s]], axis=1)
        s *= 2
    excl = a - oh
    rank = jnp.sum(oh * excl, axis=0, keepdims=True).astype(jnp.int32)
    acc = rank < cap

    flat_w = jnp.concatenate([w0, w1], axis=1)
    per_w = n_flat // SC_WORKERS
    trash = (N_EXPERTS * cap
             + lax.broadcasted_iota(jnp.int32, (1, n_flat), 1) // per_w)
    dst_ref[...] = jnp.where(acc, fe * cap + rank, trash)
    aw_ref[...] = jnp.where(acc, flat_w, 0.0)

    tot_e = a[:, n_flat - 1:n_flat]
    cnt = jnp.minimum(tot_e, float(cap))
    cnt_row = lax.dot_general(
        jnp.ones((1, 1), jnp.float32), cnt, (((1,), (1,)), ((), ())),
        preferred_element_type=jnp.float32)
    cnt_ref[...] = cnt_row.astype(jnp.int32)

    total = jnp.maximum(jnp.sum(cnt), 1.0)
    f_i = cnt / total
    p_mean = jnp.sum(probs, axis=1, keepdims=True) / float(n_tok)
    aux = AUX_COEFF * N_EXPERTS * jnp.sum(f_i * p_mean)
    aux_ref[...] = jnp.reshape(aux, (1, 1))

    xp = _pack_halves(x.astype(jnp.bfloat16))
    tcol = lax.broadcasted_iota(jnp.int32, (n_tok, 8), 0)
    xp_ref[pl.ds(0, n_tok), pl.ds(0, HALF)] = xp
    xp_ref[pl.ds(0, n_tok), pl.ds(HALF, 8)] = tcol
    xp_ref[pl.ds(n_tok, n_tok), pl.ds(0, HALF)] = xp
    xp_ref[pl.ds(n_tok, n_tok), pl.ds(HALF, 8)] = tcol + n_tok


def _ffn_body(n_j, cap, trash, x_ref, gw_ref, uw_ref, dw_ref, cnt_ref, y_ref,
              acc_ref):
    e = pl.program_id(0)
    j = pl.program_id(1)
    xb = _unpack_halves(x_ref[:, :HALF])
    gw = gw_ref[0].astype(jnp.bfloat16)
    uw = uw_ref[0].astype(jnp.bfloat16)
    dw = dw_ref[0].astype(jnp.bfloat16)
    dn = (((1,), (1,)), ((), ()))
    g = lax.dot_general(xb, gw, dn, preferred_element_type=jnp.float32)
    u = lax.dot_general(xb, uw, dn, preferred_element_type=jnp.float32)
    h = (g * (1.0 / (1.0 + jnp.exp(-g)))) * u
    y = lax.dot_general(h.astype(jnp.bfloat16), dw, dn,
                        preferred_element_type=jnp.float32)

    @pl.when(j == 0)
    def _():
        acc_ref[...] = y

    @pl.when(j != 0)
    def _():
        acc_ref[...] = acc_ref[...] + y

    @pl.when(j == n_j - 1)
    def _():
        py = _pack_halves(acc_ref[...].astype(jnp.bfloat16))
        dcol = x_ref[:, HALF:HALF + 1]
        rowc = lax.broadcasted_iota(jnp.int32, (cap, 1), 0)
        dcol = jnp.where(rowc < cnt_ref[0, e], dcol, trash)
        y_ref[0] = jnp.concatenate(
            [py, jnp.broadcast_to(dcol, (cap, ROW_W - HALF))], axis=1)


def _final_body(y0_ref, y1_ref, aw0_ref, aw1_ref, o_ref):
    aw0 = aw0_ref[...]
    aw1 = aw1_ref[...]
    y0 = _unpack_halves(y0_ref[:, :HALF]).astype(jnp.float32)
    y1 = _unpack_halves(y1_ref[:, :HALF]).astype(jnp.float32)
    o_ref[...] = (jnp.where(aw0 > 0, aw0 * y0, 0.0)
                  + jnp.where(aw1 > 0, aw1 * y1, 0.0))


def _sc_mesh():
    return plsc.VectorSubcoreMesh(
        core_axis_name="c", subcore_axis_name="s",
        num_cores=SC_CORES, num_subcores=SC_SUBCORES)


def _make_dispatch(n_flat, n_rows):
    per_w = n_flat // SC_WORKERS
    n_ch = per_w // DISP_CHUNK

    @functools.partial(
        pl.kernel,
        out_type=jax.ShapeDtypeStruct((n_rows, ROW_W), jnp.int32),
        mesh=_sc_mesh(),
        scratch_types=[
            pltpu.VMEM((NBUF, DISP_CHUNK), jnp.int32),
            pltpu.VMEM((NBUF, DISP_CHUNK, ROW_W), jnp.int32),
            [pltpu.SemaphoreType.DMA] * NBUF,
        ],
    )
    def dispatch(xp_hbm, dst_hbm, out_hbm, idx_v, rows_v, sems):
        wid = lax.axis_index("s") * SC_CORES + lax.axis_index("c")
        base = wid * per_w

        def load(i, b):
            off = base + i * DISP_CHUNK
            pltpu.sync_copy(dst_hbm.at[pl.ds(off, DISP_CHUNK)], idx_v.at[b])
            pltpu.sync_copy(xp_hbm.at[pl.ds(off, DISP_CHUNK)], rows_v.at[b])

        def scatter(b):
            return pltpu.async_copy(rows_v.at[b], out_hbm.at[idx_v.at[b]],
                                    sems[b])

        load(0, 0)
        cp = scatter(0)
        for i in range(1, n_ch):
            b = i % NBUF
            load(i, b)
            cp.wait()
            cp = scatter(b)
        cp.wait()

    return dispatch


def _make_combine(n_slots, n_out):
    per_w = n_slots // SC_WORKERS
    chunk = per_w // NBUF
    assert per_w % NBUF == 0 and chunk % 8 == 0

    @functools.partial(
        pl.kernel,
        out_type=jax.ShapeDtypeStruct((n_out, ROW_W), jnp.int32),
        mesh=_sc_mesh(),
        scratch_types=[
            pltpu.VMEM((NBUF, chunk), jnp.int32),
            pltpu.VMEM((NBUF, chunk, ROW_W), jnp.int32),
            [pltpu.SemaphoreType.DMA] * NBUF,
        ],
        compiler_params=pltpu.CompilerParams(needs_layout_passes=False),
    )
    def combine(y_hbm, out_hbm, idx_v, rows_v, sems):
        wid = lax.axis_index("s") * SC_CORES + lax.axis_index("c")
        base = wid * per_w
        lane = jnp.full((16,), HALF, jnp.int32)

        def load(i, b):
            off = base + i * chunk
            pltpu.sync_copy(y_hbm.at[pl.ds(off, chunk)], rows_v.at[b])
            for j in range(chunk // 16):
                rows16 = lax.iota(jnp.int32, 16) + 16 * j
                vals = plsc.load_gather(rows_v.at[b], [rows16, lane])
                idx_v[b, pl.ds(16 * j, 16)] = vals

        def scatter(b):
            return pltpu.async_copy(rows_v.at[b], out_hbm.at[idx_v.at[b]],
                                    sems[b])

        load(0, 0)
        cp = scatter(0)
        for i in range(1, NBUF):
            load(i, i)
            cp.wait()
            cp = scatter(i)
        cp.wait()

    return combine


def kernel(x, W_router, gate_w, up_w, down_w):
    bb, tt, d = x.shape
    n_tok = bb * tt
    n_flat = N_ACTIVE * n_tok
    cap = max(int(tt * N_ACTIVE * CAPACITY_FACTOR / N_EXPERTS), 1)
    n_rows = N_EXPERTS * cap + SC_WORKERS
    n_out = n_flat + 8

    x_flat = x.reshape(n_tok, d)

    dst, aw, cnt, aux, xp = pl.pallas_call(
        functools.partial(_router_body, cap, n_tok),
        out_shape=[
            jax.ShapeDtypeStruct((1, n_flat), jnp.int32),
            jax.ShapeDtypeStruct((1, n_flat), jnp.float32),
            jax.ShapeDtypeStruct((1, N_EXPERTS), jnp.int32),
            jax.ShapeDtypeStruct((1, 1), jnp.float32),
            jax.ShapeDtypeStruct((n_flat, ROW_W), jnp.int32),
        ],
    )(x_flat, W_router)

    buf = _make_dispatch(n_flat, n_rows)(xp, dst.reshape(n_flat))

    n_j = D_FF // F_BLK
    y = pl.pallas_call(
        functools.partial(_ffn_body, n_j, cap, n_flat),
        grid=(N_EXPERTS, n_j),
        in_specs=[
            pl.BlockSpec((cap, ROW_W), lambda e, j: (e, 0)),
            pl.BlockSpec((1, F_BLK, d), lambda e, j: (e, j, 0)),
            pl.BlockSpec((1, F_BLK, d), lambda e, j: (e, j, 0)),
            pl.BlockSpec((1, d, F_BLK), lambda e, j: (e, 0, j)),
            pl.BlockSpec(memory_space=pltpu.SMEM),
        ],
        out_specs=pl.BlockSpec((1, cap, ROW_W), lambda e, j: (e, 0, 0)),
        out_shape=jax.ShapeDtypeStruct((N_EXPERTS, cap, ROW_W), jnp.int32),
        scratch_shapes=[pltpu.VMEM((cap, d), jnp.float32)],
        compiler_params=pltpu.CompilerParams(
            dimension_semantics=("parallel", "arbitrary")),
    )(buf, gate_w, up_w, down_w, cnt)

    yg = _make_combine(N_EXPERTS * cap, n_out)(
        y.reshape(N_EXPERTS * cap, ROW_W))

    aw_col = aw.reshape(n_flat, 1)
    blk = 512
    out_flat = pl.pallas_call(
        _final_body,
        grid=(n_tok // blk,),
        in_specs=[
            pl.BlockSpec((blk, ROW_W), lambda i: (i, 0)),
            pl.BlockSpec((blk, ROW_W), lambda i, n=n_tok // blk: (i + n, 0)),
            pl.BlockSpec((blk, 1), lambda i: (i, 0)),
            pl.BlockSpec((blk, 1), lambda i, n=n_tok // blk: (i + n, 0)),
        ],
        out_specs=pl.BlockSpec((blk, d), lambda i: (i, 0)),
        out_shape=jax.ShapeDtypeStruct((n_tok, d), jnp.float32),
        compiler_params=pltpu.CompilerParams(
            dimension_semantics=("parallel",)),
    )(yg, yg, aw_col, aw_col)

    return out_flat.reshape(bb, tt, d), aux.reshape(())
